# baseline (speedup 1.0000x reference)
import functools

import jax
import jax.numpy as jnp
from jax import lax
from jax.experimental import pallas as pl
from jax.experimental.pallas import tpu as pltpu
from jax.experimental.pallas import tpu_sc as plsc

N_NODES = 10000
N_EDGES = 320000
B = 64

_NC = 2
_NS = 16
_NW = _NC * _NS


def _make_sc_seg_sum(n, w, e, compute_deg, halves):
    C = 80
    SEG = 16 if halves else 8
    NSEG = 16
    P = SEG // 2
    rps = 1000
    n_drain = n // rps

    BR = 200
    n_bounce = rps // BR

    mesh = plsc.VectorSubcoreMesh(core_axis_name="c", subcore_axis_name="s")
    out_type = [jax.ShapeDtypeStruct((_NC, n, w), jnp.float32)]
    scratch = [
        pltpu.VMEM_SHARED((n + 8, w), jnp.float32),
        pltpu.VMEM((SEG, C), jnp.int32),
        pltpu.VMEM((SEG, C), jnp.int32),
        pltpu.VMEM((C, w), jnp.float32),
        pltpu.VMEM((C, w), jnp.float32),
        pltpu.VMEM((BR, w), jnp.float32),
        pltpu.VMEM((C,), jnp.int32),
        pltpu.SemaphoreType.DMA,
        pltpu.SemaphoreType.DMA,
    ]
    if compute_deg:
        out_type.append(jax.ShapeDtypeStruct((_NC * n,), jnp.float32))
        scratch += [
            pltpu.VMEM_SHARED((n + 8,), jnp.float32),
            pltpu.VMEM((C,), jnp.float32),
            pltpu.VMEM((rps,), jnp.float32),
        ]

    @functools.partial(
        pl.kernel, mesh=mesh, out_type=tuple(out_type), scratch_types=scratch)
    def k(ha_hbm, hb_hbm, src_hbm, dst_hbm, zrow_hbm, zdeg_hbm, *rest):
        if compute_deg:
            (acc_out, deg_out, shared_acc, src_tab, dst_tab, buf0, buf1,
             zbuf, dst_v, sem0, sem1, shared_deg, ones_v, degbuf) = rest
        else:
            (acc_out, shared_acc, src_tab, dst_tab, buf0, buf1, zbuf,
             dst_v, sem0, sem1) = rest
        cid = lax.axis_index("c")
        sid = lax.axis_index("s")
        wid = sid * _NC + cid
        tab_id = sid if halves else wid

        @pl.when(sid < n_drain)
        def _():
            pltpu.sync_copy(zrow_hbm, zbuf)
            for t in range(n_bounce):
                pltpu.sync_copy(zbuf,
                                shared_acc.at[pl.ds(sid * rps + t * BR, BR)])
        if compute_deg:
            @pl.when(sid < n_drain)
            def _():
                pltpu.sync_copy(zdeg_hbm, degbuf)
                pltpu.sync_copy(degbuf, shared_deg.at[pl.ds(sid * rps, rps)])
            for i in range(C // 16):
                ones_v[pl.ds(i * 16, 16)] = jnp.full((16,), 1.0, jnp.float32)
        plsc.subcore_barrier()

        def run_edges(h_hbm):
            def seg(s, carry):
                pltpu.sync_copy(src_hbm.at[tab_id, s], src_tab)
                pltpu.sync_copy(dst_hbm.at[tab_id, s], dst_tab)
                pltpu.async_copy(h_hbm.at[src_tab.at[0]], buf0, sem0)

                def pair(i, carry2):
                    j0 = 2 * i
                    pltpu.async_copy(h_hbm.at[src_tab.at[j0 + 1]], buf1, sem1)
                    for q in range(C // 16):
                        dst_v[pl.ds(q * 16, 16)] = dst_tab[j0,
                                                           pl.ds(q * 16, 16)]
                    pltpu.make_async_copy(h_hbm.at[src_tab.at[j0]], buf0,
                                          sem0).wait()
                    pltpu.sync_copy(buf0, shared_acc.at[dst_v], add=True)
                    if compute_deg:
                        pltpu.sync_copy(ones_v, shared_deg.at[dst_v],
                                        add=True)

                    @pl.when(i < P - 1)
                    def _():
                        pltpu.async_copy(h_hbm.at[src_tab.at[j0 + 2]], buf0,
                                         sem0)

                    for q in range(C // 16):
                        dst_v[pl.ds(q * 16, 16)] = dst_tab[j0 + 1,
                                                           pl.ds(q * 16, 16)]
                    pltpu.make_async_copy(h_hbm.at[src_tab.at[j0 + 1]], buf1,
                                          sem1).wait()
                    pltpu.sync_copy(buf1, shared_acc.at[dst_v], add=True)
                    if compute_deg:
                        pltpu.sync_copy(ones_v, shared_deg.at[dst_v],
                                        add=True)
                    return carry2

                lax.fori_loop(0, P, pair, 0)
                return carry

            lax.fori_loop(0, NSEG, seg, 0)

        if halves:
            @pl.when(cid == 0)
            def _():
                run_edges(ha_hbm)

            @pl.when(cid == 1)
            def _():
                run_edges(hb_hbm)
        else:
            run_edges(ha_hbm)
        plsc.subcore_barrier()

        r0 = sid * rps

        @pl.when(sid < n_drain)
        def _():
            for t in range(n_bounce):
                pltpu.sync_copy(shared_acc.at[pl.ds(r0 + t * BR, BR)], zbuf)
                pltpu.sync_copy(zbuf, acc_out.at[cid, pl.ds(r0 + t * BR, BR)])
            if compute_deg:
                pltpu.sync_copy(shared_deg.at[pl.ds(r0, rps)], degbuf)
                pltpu.sync_copy(degbuf, deg_out.at[pl.ds(cid * n + r0, rps)])

    return k


def _pad_tables(src, dst, n, nworkers, seg):
    e = src.shape[0]
    epw = e // nworkers
    pad = 16 * seg * 80 - epw
    s2 = src.reshape(nworkers, epw)
    d2 = dst.reshape(nworkers, epw)
    s2 = jnp.concatenate(
        [s2, jnp.zeros((nworkers, pad), jnp.int32)], axis=1)
    d2 = jnp.concatenate(
        [d2, jnp.full((nworkers, pad), n, jnp.int32)], axis=1)
    return (s2.reshape(nworkers, 16, seg, 80),
            d2.reshape(nworkers, 16, seg, 80))


def _sc_seg_sum(h, src, dst, compute_deg):
    n, w = h.shape
    zrow = jnp.zeros((200, w), jnp.float32)
    zdeg = jnp.zeros((1000,), jnp.float32)
    src4, dst4 = _pad_tables(src, dst, n, _NW, 8)
    k = _make_sc_seg_sum(n, w, src.shape[0], compute_deg, halves=False)
    return k(h, h, src4, dst4, zrow, zdeg)


def _sc_seg_sum_halves(ha, hb, src, dst):
    n, w = ha.shape
    zrow = jnp.zeros((200, w), jnp.float32)
    zdeg = jnp.zeros((1000,), jnp.float32)
    src4, dst4 = _pad_tables(src, dst, n, _NS, 16)
    k = _make_sc_seg_sum(n, w, src.shape[0], compute_deg=False, halves=True)
    return k(ha, hb, src4, dst4, zrow, zdeg)


_RB = 400


def _tc_layer1(acc, deg, x, WlT, WrT, b):
    n, w = x.shape
    dout = WlT.shape[1]
    grid = (n // _RB,)

    def body(acc_ref, deg_ref, x_ref, wl_ref, wr_ref, b_ref, h_ref, inv_ref):
        d = jnp.maximum(deg_ref[0] + deg_ref[1], 1.0)
        inv = 1.0 / d
        inv_ref[...] = inv
        agg = (acc_ref[0] + acc_ref[1]) * inv
        h = jnp.dot(agg, wl_ref[...], preferred_element_type=jnp.float32)
        h += jnp.dot(x_ref[...], wr_ref[...], preferred_element_type=jnp.float32)
        h_ref[...] = jnp.maximum(h + b_ref[...], 0.0)

    return pl.pallas_call(
        body,
        grid=grid,
        in_specs=[
            pl.BlockSpec((2, _RB, w), lambda i: (0, i, 0)),
            pl.BlockSpec((2, _RB, 1), lambda i: (0, i, 0)),
            pl.BlockSpec((_RB, w), lambda i: (i, 0)),
            pl.BlockSpec(WlT.shape, lambda i: (0, 0)),
            pl.BlockSpec(WrT.shape, lambda i: (0, 0)),
            pl.BlockSpec((1, dout), lambda i: (0, 0)),
        ],
        out_specs=[
            pl.BlockSpec((_RB, dout), lambda i: (i, 0)),
            pl.BlockSpec((_RB, 1), lambda i: (i, 0)),
        ],
        out_shape=[
            jax.ShapeDtypeStruct((n, dout), jnp.float32),
            jax.ShapeDtypeStruct((n, 1), jnp.float32),
        ],
    )(acc, deg.reshape(2, n, 1), x, WlT, WrT, b.reshape(1, dout))


def _tc_layer(acc, inv_deg, x, WlT, WrT, b, merge):
    n = x.shape[0]
    din = x.shape[1]
    dout = WlT.shape[1]
    w_each = acc.shape[2]
    grid = (n // _RB,)

    def body(acc_ref, inv_ref, x_ref, wl_ref, wr_ref, b_ref, h_ref):
        inv = inv_ref[...]
        if merge:
            agg = (acc_ref[0] + acc_ref[1]) * inv
        else:
            agg = jnp.concatenate([acc_ref[0] * inv, acc_ref[1] * inv], axis=1)
        h = jnp.dot(agg, wl_ref[...], preferred_element_type=jnp.float32)
        h += jnp.dot(x_ref[...], wr_ref[...], preferred_element_type=jnp.float32)
        h_ref[...] = jnp.maximum(h + b_ref[...], 0.0)

    in_specs = [pl.BlockSpec((2, _RB, w_each), lambda i: (0, i, 0))]
    in_specs += [
        pl.BlockSpec((_RB, 1), lambda i: (i, 0)),
        pl.BlockSpec((_RB, din), lambda i: (i, 0)),
        pl.BlockSpec(WlT.shape, lambda i: (0, 0)),
        pl.BlockSpec(WrT.shape, lambda i: (0, 0)),
        pl.BlockSpec((1, dout), lambda i: (0, 0)),
    ]
    return pl.pallas_call(
        body,
        grid=grid,
        in_specs=in_specs,
        out_specs=pl.BlockSpec((_RB, dout), lambda i: (i, 0)),
        out_shape=jax.ShapeDtypeStruct((n, dout), jnp.float32),
    )(acc, inv_deg, x, WlT, WrT, b.reshape(1, dout))


def _tc_segment_max(h, batch2d, nseg):
    n, d = h.shape
    grid = (n // _RB,)

    def body(h_ref, b_ref, g_ref):
        i = pl.program_id(0)

        @pl.when(i == 0)
        def _():
            g_ref[...] = jnp.full((nseg, d), -jnp.inf, jnp.float32)

        bmin = b_ref[0, 0]
        bmax = b_ref[_RB - 1, 0]
        hv = h_ref[...]
        bv = b_ref[...]
        for s in range(nseg):
            @pl.when((s >= bmin) & (s <= bmax))
            def _():
                vals = jnp.where(bv == s, hv, -jnp.inf)
                m = jnp.max(vals, axis=0, keepdims=True)
                g_ref[pl.ds(s, 1), :] = jnp.maximum(g_ref[pl.ds(s, 1), :], m)

    return pl.pallas_call(
        body,
        grid=grid,
        in_specs=[
            pl.BlockSpec((_RB, d), lambda i: (i, 0)),
            pl.BlockSpec((_RB, 1), lambda i: (i, 0)),
        ],
        out_specs=pl.BlockSpec((nseg, d), lambda i: (0, 0)),
        out_shape=jax.ShapeDtypeStruct((nseg, d), jnp.float32),
    )(h, batch2d)


def _tc_conv_stage1(x3, w1, b1):
    L = x3.shape[2]
    lo = L - 7
    lp = (lo // 3)
    cout = w1.shape[0]

    def body(x_ref, w_ref, b_ref, o_ref):
        xr = x_ref[...].reshape(1, L)
        cols = jnp.concatenate([xr[:, t:t + lo] for t in range(8)], axis=0)
        pre = jnp.dot(w_ref[...], cols, preferred_element_type=jnp.float32)
        pre = jnp.maximum(pre + b_ref[...], 0.0)
        preT = jnp.transpose(pre, (1, 0))
        preT = preT[:lp * 3].reshape(lp, 3, cout)
        o_ref[...] = jnp.max(preT, axis=1)[None]

    return pl.pallas_call(
        body,
        grid=(B,),
        in_specs=[
            pl.BlockSpec((1, 1, L), lambda n: (n, 0, 0)),
            pl.BlockSpec((cout, 8), lambda n: (0, 0)),
            pl.BlockSpec((cout, 1), lambda n: (0, 0)),
        ],
        out_specs=pl.BlockSpec((1, lp, cout), lambda n: (n, 0, 0)),
        out_shape=jax.ShapeDtypeStruct((B, lp, cout), jnp.float32),
    )(x3, w1, b1.reshape(cout, 1))


def _tc_conv_stage(x, wT, b):
    _, L, cin = x.shape
    cout = wT.shape[2]
    lo = L - 7
    lp = lo // 3

    def body(x_ref, w_ref, b_ref, o_ref):
        xc = x_ref[...].reshape(L, cin)
        pre = b_ref[...]
        for t in range(8):
            pre = pre + jnp.dot(xc[t:t + lo, :], w_ref[t],
                                preferred_element_type=jnp.float32)
        pre = jnp.maximum(pre, 0.0)
        pre = pre[:lp * 3].reshape(lp, 3, cout)
        o_ref[...] = jnp.max(pre, axis=1)[None]

    return pl.pallas_call(
        body,
        grid=(B,),
        in_specs=[
            pl.BlockSpec((1, L, cin), lambda n: (n, 0, 0)),
            pl.BlockSpec(wT.shape, lambda n: (0, 0, 0)),
            pl.BlockSpec((1, cout), lambda n: (0, 0)),
        ],
        out_specs=pl.BlockSpec((1, lp, cout), lambda n: (n, 0, 0)),
        out_shape=jax.ShapeDtypeStruct((B, lp, cout), jnp.float32),
    )(x, wT, b.reshape(1, cout))


def _tc_proj(flat, WxtT, bxt):
    k = flat.shape[1]
    kb = 8832
    steps = k // kb
    dout = WxtT.shape[1]

    def body(a_ref, w_ref, b_ref, o_ref):
        j = pl.program_id(0)

        @pl.when(j == 0)
        def _():
            o_ref[...] = jnp.broadcast_to(b_ref[...], (B, dout))

        o_ref[...] += jnp.dot(a_ref[...], w_ref[...],
                              preferred_element_type=jnp.float32)

    return pl.pallas_call(
        body,
        grid=(steps,),
        in_specs=[
            pl.BlockSpec((B, kb), lambda j: (0, j)),
            pl.BlockSpec((kb, dout), lambda j: (j, 0)),
            pl.BlockSpec((1, dout), lambda j: (0, 0)),
        ],
        out_specs=pl.BlockSpec((B, dout), lambda j: (0, 0)),
        out_shape=jax.ShapeDtypeStruct((B, dout), jnp.float32),
    )(flat, WxtT, bxt.reshape(1, dout))


def _tc_heads(g_raw, xt, Wg1T, bg1, Wg2T, bg2, Wf1T, bf1, Wf2T, bf2,
              WoutT, bout):
    def body(g_ref, xt_ref, wg1, bg1r, wg2, bg2r, wf1, bf1r, wf2, bf2r,
             wo, bor, o_ref):
        g = jnp.maximum(jnp.dot(g_ref[...], wg1[...],
                                preferred_element_type=jnp.float32)
                        + bg1r[...], 0.0)
        g = jnp.dot(g, wg2[...], preferred_element_type=jnp.float32) + bg2r[...]
        xc = jnp.concatenate([g, xt_ref[...]], axis=1)
        f = jnp.maximum(jnp.dot(xc, wf1[...],
                                preferred_element_type=jnp.float32)
                        + bf1r[...], 0.0)
        f = jnp.maximum(jnp.dot(f, wf2[...],
                                preferred_element_type=jnp.float32)
                        + bf2r[...], 0.0)
        z = jnp.dot(f, wo[...], preferred_element_type=jnp.float32) + bor[...]
        o_ref[...] = jax.nn.sigmoid(z)

    args = [g_raw, xt, Wg1T, bg1.reshape(1, -1), Wg2T, bg2.reshape(1, -1),
            Wf1T, bf1.reshape(1, -1), Wf2T, bf2.reshape(1, -1),
            WoutT, bout.reshape(1, -1)]
    return pl.pallas_call(
        body,
        out_shape=jax.ShapeDtypeStruct((B, 1), jnp.float32),
    )(*args)


def kernel(x, edge_index, batch, x_cell_mut, edge_feat, W_l1, W_r1, b1,
           W_l2, W_r2, b2, W_l3, W_r3, b3, Wg1, bg1, Wg2, bg2, Wc1, bc1,
           Wc2, bc2, Wc3, bc3, Wxt, bxt, Wf1, bf1, Wf2, bf2, Wout, bout):
    src = edge_index[0]
    dst = edge_index[1]

    acc1, deg = _sc_seg_sum(x, src, dst, compute_deg=True)
    h1, inv_deg = _tc_layer1(acc1, deg, x, W_l1.T, W_r1.T, b1)

    acc2 = _sc_seg_sum(h1, src, dst, compute_deg=False)[0]
    h2 = _tc_layer(acc2, inv_deg, h1, W_l2.T, W_r2.T, b2, merge=True)

    acc3 = _sc_seg_sum_halves(h2[:, :128], h2[:, 128:], src, dst)[0]
    h3 = _tc_layer(acc3, inv_deg, h2, W_l3.T, W_r3.T, b3, merge=False)

    g_raw = _tc_segment_max(h3, batch.reshape(N_NODES, 1), B)

    c1 = _tc_conv_stage1(x_cell_mut, Wc1.reshape(32, 8), bc1)
    w2T = jnp.transpose(Wc2, (2, 1, 0))
    c2 = _tc_conv_stage(c1, w2T, bc2)
    w3T = jnp.transpose(Wc3, (2, 1, 0))
    c3 = _tc_conv_stage(c2, w3T, bc3)

    flat = c3.reshape(B, -1)
    WxtT = jnp.transpose(Wxt.reshape(128, 128, 483), (2, 1, 0)).reshape(61824, 128)
    xt = _tc_proj(flat, WxtT, bxt)

    return _tc_heads(g_raw, xt, Wg1.T, bg1, Wg2.T, bg2, Wf1.T, bf1,
                     Wf2.T, bf2, Wout.T, bout)


# speedup vs baseline: 1.1676x; 1.1676x over previous
import functools

import jax
import jax.numpy as jnp
from jax import lax
from jax.experimental import pallas as pl
from jax.experimental.pallas import tpu as pltpu
from jax.experimental.pallas import tpu_sc as plsc

N_NODES = 10000
N_EDGES = 320000
B = 64

_NC = 2
_NS = 16
_NW = _NC * _NS


def _make_sc_seg_sum(n, w, e, compute_deg, halves):
    C = 80
    SEG = 16 if halves else 8
    NSEG = 16
    P = SEG // 2
    rps = 1000
    n_drain = n // rps

    BR = 200
    n_bounce = rps // BR

    mesh = plsc.VectorSubcoreMesh(core_axis_name="c", subcore_axis_name="s")
    out_type = [jax.ShapeDtypeStruct((_NC, n, w), jnp.float32)]
    scratch = [
        pltpu.VMEM_SHARED((n + 8, w), jnp.float32),
        pltpu.VMEM((SEG, C), jnp.int32),
        pltpu.VMEM((SEG, C), jnp.int32),
        pltpu.VMEM((C, w), jnp.float32),
        pltpu.VMEM((C, w), jnp.float32),
        pltpu.VMEM((BR, w), jnp.float32),
        pltpu.VMEM((C,), jnp.int32),
        pltpu.SemaphoreType.DMA,
        pltpu.SemaphoreType.DMA,
    ]
    if compute_deg:
        out_type.append(jax.ShapeDtypeStruct((_NC * n,), jnp.float32))
        scratch += [
            pltpu.VMEM_SHARED((n + 8,), jnp.float32),
            pltpu.VMEM((C,), jnp.float32),
            pltpu.VMEM((rps,), jnp.float32),
        ]

    @functools.partial(
        pl.kernel, mesh=mesh, out_type=tuple(out_type), scratch_types=scratch)
    def k(ha_hbm, hb_hbm, src_hbm, dst_hbm, zrow_hbm, zdeg_hbm, *rest):
        if compute_deg:
            (acc_out, deg_out, shared_acc, src_tab, dst_tab, buf0, buf1,
             zbuf, dst_v, sem0, sem1, shared_deg, ones_v, degbuf) = rest
        else:
            (acc_out, shared_acc, src_tab, dst_tab, buf0, buf1, zbuf,
             dst_v, sem0, sem1) = rest
        cid = lax.axis_index("c")
        sid = lax.axis_index("s")
        wid = sid * _NC + cid
        tab_id = sid if halves else wid

        @pl.when(sid < n_drain)
        def _():
            pltpu.sync_copy(zrow_hbm, zbuf)
            for t in range(n_bounce):
                pltpu.sync_copy(zbuf,
                                shared_acc.at[pl.ds(sid * rps + t * BR, BR)])
        if compute_deg:
            @pl.when(sid < n_drain)
            def _():
                pltpu.sync_copy(zdeg_hbm, degbuf)
                pltpu.sync_copy(degbuf, shared_deg.at[pl.ds(sid * rps, rps)])
            for i in range(C // 16):
                ones_v[pl.ds(i * 16, 16)] = jnp.full((16,), 1.0, jnp.float32)
        plsc.subcore_barrier()

        def run_edges(h_hbm):
            def seg(s, carry):
                pltpu.sync_copy(src_hbm.at[tab_id, s], src_tab)
                pltpu.sync_copy(dst_hbm.at[tab_id, s], dst_tab)
                pltpu.async_copy(h_hbm.at[src_tab.at[0]], buf0, sem0)

                def pair(i, carry2):
                    j0 = 2 * i
                    pltpu.async_copy(h_hbm.at[src_tab.at[j0 + 1]], buf1, sem1)
                    for q in range(C // 16):
                        dst_v[pl.ds(q * 16, 16)] = dst_tab[j0,
                                                           pl.ds(q * 16, 16)]
                    pltpu.make_async_copy(h_hbm.at[src_tab.at[j0]], buf0,
                                          sem0).wait()
                    pltpu.sync_copy(buf0, shared_acc.at[dst_v], add=True)
                    if compute_deg:
                        pltpu.sync_copy(ones_v, shared_deg.at[dst_v],
                                        add=True)

                    @pl.when(i < P - 1)
                    def _():
                        pltpu.async_copy(h_hbm.at[src_tab.at[j0 + 2]], buf0,
                                         sem0)

                    for q in range(C // 16):
                        dst_v[pl.ds(q * 16, 16)] = dst_tab[j0 + 1,
                                                           pl.ds(q * 16, 16)]
                    pltpu.make_async_copy(h_hbm.at[src_tab.at[j0 + 1]], buf1,
                                          sem1).wait()
                    pltpu.sync_copy(buf1, shared_acc.at[dst_v], add=True)
                    if compute_deg:
                        pltpu.sync_copy(ones_v, shared_deg.at[dst_v],
                                        add=True)
                    return carry2

                lax.fori_loop(0, P, pair, 0)
                return carry

            lax.fori_loop(0, NSEG, seg, 0)

        if halves:
            @pl.when(cid == 0)
            def _():
                run_edges(ha_hbm)

            @pl.when(cid == 1)
            def _():
                run_edges(hb_hbm)
        else:
            run_edges(ha_hbm)
        plsc.subcore_barrier()

        r0 = sid * rps

        @pl.when(sid < n_drain)
        def _():
            for t in range(n_bounce):
                pltpu.sync_copy(shared_acc.at[pl.ds(r0 + t * BR, BR)], zbuf)
                pltpu.sync_copy(zbuf, acc_out.at[cid, pl.ds(r0 + t * BR, BR)])
            if compute_deg:
                pltpu.sync_copy(shared_deg.at[pl.ds(r0, rps)], degbuf)
                pltpu.sync_copy(degbuf, deg_out.at[pl.ds(cid * n + r0, rps)])

    return k


def _pad_tables(src, dst, n, nworkers, seg):
    e = src.shape[0]
    epw = e // nworkers
    pad = 16 * seg * 80 - epw
    s2 = src.reshape(nworkers, epw)
    d2 = dst.reshape(nworkers, epw)
    s2 = jnp.concatenate(
        [s2, jnp.zeros((nworkers, pad), jnp.int32)], axis=1)
    d2 = jnp.concatenate(
        [d2, jnp.full((nworkers, pad), n, jnp.int32)], axis=1)
    return (s2.reshape(nworkers, 16, seg, 80),
            d2.reshape(nworkers, 16, seg, 80))


def _sc_seg_sum(h, src, dst, compute_deg):
    n, w = h.shape
    zrow = jnp.zeros((200, w), jnp.float32)
    zdeg = jnp.zeros((1000,), jnp.float32)
    src4, dst4 = _pad_tables(src, dst, n, _NW, 8)
    k = _make_sc_seg_sum(n, w, src.shape[0], compute_deg, halves=False)
    return k(h, h, src4, dst4, zrow, zdeg)


def _sc_seg_sum_halves(ha, hb, src, dst):
    n, w = ha.shape
    zrow = jnp.zeros((200, w), jnp.float32)
    zdeg = jnp.zeros((1000,), jnp.float32)
    src4, dst4 = _pad_tables(src, dst, n, _NS, 16)
    k = _make_sc_seg_sum(n, w, src.shape[0], compute_deg=False, halves=True)
    return k(ha, hb, src4, dst4, zrow, zdeg)


_RB = 400


def _tc_layer1(acc, deg, x, WlT, WrT, b):
    n, w = x.shape
    dout = WlT.shape[1]
    grid = (n // _RB,)

    def body(acc_ref, deg_ref, x_ref, wl_ref, wr_ref, b_ref, h_ref, inv_ref):
        d = jnp.maximum(deg_ref[0] + deg_ref[1], 1.0)
        inv = 1.0 / d
        inv_ref[...] = inv
        agg = (acc_ref[0] + acc_ref[1]) * inv
        h = jnp.dot(agg, wl_ref[...], preferred_element_type=jnp.float32)
        h += jnp.dot(x_ref[...], wr_ref[...], preferred_element_type=jnp.float32)
        h_ref[...] = jnp.maximum(h + b_ref[...], 0.0)

    return pl.pallas_call(
        body,
        grid=grid,
        in_specs=[
            pl.BlockSpec((2, _RB, w), lambda i: (0, i, 0)),
            pl.BlockSpec((2, _RB, 1), lambda i: (0, i, 0)),
            pl.BlockSpec((_RB, w), lambda i: (i, 0)),
            pl.BlockSpec(WlT.shape, lambda i: (0, 0)),
            pl.BlockSpec(WrT.shape, lambda i: (0, 0)),
            pl.BlockSpec((1, dout), lambda i: (0, 0)),
        ],
        out_specs=[
            pl.BlockSpec((_RB, dout), lambda i: (i, 0)),
            pl.BlockSpec((_RB, 1), lambda i: (i, 0)),
        ],
        out_shape=[
            jax.ShapeDtypeStruct((n, dout), jnp.float32),
            jax.ShapeDtypeStruct((n, 1), jnp.float32),
        ],
    )(acc, deg.reshape(2, n, 1), x, WlT, WrT, b.reshape(1, dout))


def _tc_layer(acc, inv_deg, x, WlT, WrT, b, merge):
    n = x.shape[0]
    din = x.shape[1]
    dout = WlT.shape[1]
    w_each = acc.shape[2]
    grid = (n // _RB,)

    def body(acc_ref, inv_ref, x_ref, wl_ref, wr_ref, b_ref, h_ref):
        inv = inv_ref[...]
        if merge:
            agg = (acc_ref[0] + acc_ref[1]) * inv
        else:
            agg = jnp.concatenate([acc_ref[0] * inv, acc_ref[1] * inv], axis=1)
        h = jnp.dot(agg, wl_ref[...], preferred_element_type=jnp.float32)
        h += jnp.dot(x_ref[...], wr_ref[...], preferred_element_type=jnp.float32)
        h_ref[...] = jnp.maximum(h + b_ref[...], 0.0)

    in_specs = [pl.BlockSpec((2, _RB, w_each), lambda i: (0, i, 0))]
    in_specs += [
        pl.BlockSpec((_RB, 1), lambda i: (i, 0)),
        pl.BlockSpec((_RB, din), lambda i: (i, 0)),
        pl.BlockSpec(WlT.shape, lambda i: (0, 0)),
        pl.BlockSpec(WrT.shape, lambda i: (0, 0)),
        pl.BlockSpec((1, dout), lambda i: (0, 0)),
    ]
    return pl.pallas_call(
        body,
        grid=grid,
        in_specs=in_specs,
        out_specs=pl.BlockSpec((_RB, dout), lambda i: (i, 0)),
        out_shape=jax.ShapeDtypeStruct((n, dout), jnp.float32),
    )(acc, inv_deg, x, WlT, WrT, b.reshape(1, dout))


def _tc_segment_max(h, batch2d, nseg):
    n, d = h.shape
    grid = (n // _RB,)

    def body(h_ref, b_ref, g_ref):
        i = pl.program_id(0)

        @pl.when(i == 0)
        def _():
            g_ref[...] = jnp.full((nseg, d), -jnp.inf, jnp.float32)

        bmin = b_ref[0, 0]
        bmax = b_ref[_RB - 1, 0]
        hv = h_ref[...]
        bv = b_ref[...]
        for s in range(nseg):
            @pl.when((s >= bmin) & (s <= bmax))
            def _():
                vals = jnp.where(bv == s, hv, -jnp.inf)
                m = jnp.max(vals, axis=0, keepdims=True)
                g_ref[pl.ds(s, 1), :] = jnp.maximum(g_ref[pl.ds(s, 1), :], m)

    return pl.pallas_call(
        body,
        grid=grid,
        in_specs=[
            pl.BlockSpec((_RB, d), lambda i: (i, 0)),
            pl.BlockSpec((_RB, 1), lambda i: (i, 0)),
        ],
        out_specs=pl.BlockSpec((nseg, d), lambda i: (0, 0)),
        out_shape=jax.ShapeDtypeStruct((nseg, d), jnp.float32),
    )(h, batch2d)


def _phase_weights(w):
    cout, cin, _ = w.shape
    rows = []
    for rr in range(12):
        blocks = []
        for j in range(3):
            t = rr - j
            if 0 <= t < 8:
                blocks.append(w[:, :, t].T)
            else:
                blocks.append(jnp.zeros((cin, cout), jnp.float32))
        rows.append(jnp.concatenate(blocks, axis=1))
    return jnp.concatenate(rows, axis=0)


def _tc_conv_phase(xg, wp, bp, lp, cout):
    _, G, M = xg.shape

    def body(x_ref, w_ref, b_ref, o_ref):
        xr = x_ref[...].reshape(G, M)
        cols = jnp.concatenate([xr[s:s + lp] for s in range(4)], axis=1)
        pre = jnp.dot(cols, w_ref[...], preferred_element_type=jnp.float32)
        pre = jnp.maximum(pre + b_ref[...], 0.0)
        y = jnp.maximum(jnp.maximum(pre[:, :cout], pre[:, cout:2 * cout]),
                        pre[:, 2 * cout:])
        o_ref[...] = y[None]

    return pl.pallas_call(
        body,
        grid=(B,),
        in_specs=[
            pl.BlockSpec((1, G, M), lambda n: (n, 0, 0)),
            pl.BlockSpec(wp.shape, lambda n: (0, 0)),
            pl.BlockSpec((1, 3 * cout), lambda n: (0, 0)),
        ],
        out_specs=pl.BlockSpec((1, lp, cout), lambda n: (n, 0, 0)),
        out_shape=jax.ShapeDtypeStruct((B, lp, cout), jnp.float32),
    )(xg, wp, bp)


def _tc_conv_stage1(x3, w1, b1):
    L = x3.shape[2]
    lo = L - 7
    lp = (lo // 3)
    cout = w1.shape[0]

    def body(x_ref, w_ref, b_ref, o_ref):
        xr = x_ref[...].reshape(1, L)
        cols = jnp.concatenate([xr[:, t:t + lo] for t in range(8)], axis=0)
        pre = jnp.dot(w_ref[...], cols, preferred_element_type=jnp.float32)
        pre = jnp.maximum(pre + b_ref[...], 0.0)
        preT = jnp.transpose(pre, (1, 0))
        preT = preT[:lp * 3].reshape(lp, 3, cout)
        o_ref[...] = jnp.max(preT, axis=1)[None]

    return pl.pallas_call(
        body,
        grid=(B,),
        in_specs=[
            pl.BlockSpec((1, 1, L), lambda n: (n, 0, 0)),
            pl.BlockSpec((cout, 8), lambda n: (0, 0)),
            pl.BlockSpec((cout, 1), lambda n: (0, 0)),
        ],
        out_specs=pl.BlockSpec((1, lp, cout), lambda n: (n, 0, 0)),
        out_shape=jax.ShapeDtypeStruct((B, lp, cout), jnp.float32),
    )(x3, w1, b1.reshape(cout, 1))


def _tc_conv_stage(x, wT, b):
    _, L, cin = x.shape
    cout = wT.shape[2]
    lo = L - 7
    lp = lo // 3

    def body(x_ref, w_ref, b_ref, o_ref):
        xc = x_ref[...].reshape(L, cin)
        pre = b_ref[...]
        for t in range(8):
            pre = pre + jnp.dot(xc[t:t + lo, :], w_ref[t],
                                preferred_element_type=jnp.float32)
        pre = jnp.maximum(pre, 0.0)
        pre = pre[:lp * 3].reshape(lp, 3, cout)
        o_ref[...] = jnp.max(pre, axis=1)[None]

    return pl.pallas_call(
        body,
        grid=(B,),
        in_specs=[
            pl.BlockSpec((1, L, cin), lambda n: (n, 0, 0)),
            pl.BlockSpec(wT.shape, lambda n: (0, 0, 0)),
            pl.BlockSpec((1, cout), lambda n: (0, 0)),
        ],
        out_specs=pl.BlockSpec((1, lp, cout), lambda n: (n, 0, 0)),
        out_shape=jax.ShapeDtypeStruct((B, lp, cout), jnp.float32),
    )(x, wT, b.reshape(1, cout))


def _tc_proj(flat, WxtT, bxt):
    k = flat.shape[1]
    kb = 8832
    steps = k // kb
    dout = WxtT.shape[1]

    def body(a_ref, w_ref, b_ref, o_ref):
        j = pl.program_id(0)

        @pl.when(j == 0)
        def _():
            o_ref[...] = jnp.broadcast_to(b_ref[...], (B, dout))

        o_ref[...] += jnp.dot(a_ref[...], w_ref[...],
                              preferred_element_type=jnp.float32)

    return pl.pallas_call(
        body,
        grid=(steps,),
        in_specs=[
            pl.BlockSpec((B, kb), lambda j: (0, j)),
            pl.BlockSpec((kb, dout), lambda j: (j, 0)),
            pl.BlockSpec((1, dout), lambda j: (0, 0)),
        ],
        out_specs=pl.BlockSpec((B, dout), lambda j: (0, 0)),
        out_shape=jax.ShapeDtypeStruct((B, dout), jnp.float32),
    )(flat, WxtT, bxt.reshape(1, dout))


def _tc_heads(g_raw, xt, Wg1T, bg1, Wg2T, bg2, Wf1T, bf1, Wf2T, bf2,
              WoutT, bout):
    def body(g_ref, xt_ref, wg1, bg1r, wg2, bg2r, wf1, bf1r, wf2, bf2r,
             wo, bor, o_ref):
        g = jnp.maximum(jnp.dot(g_ref[...], wg1[...],
                                preferred_element_type=jnp.float32)
                        + bg1r[...], 0.0)
        g = jnp.dot(g, wg2[...], preferred_element_type=jnp.float32) + bg2r[...]
        xc = jnp.concatenate([g, xt_ref[...]], axis=1)
        f = jnp.maximum(jnp.dot(xc, wf1[...],
                                preferred_element_type=jnp.float32)
                        + bf1r[...], 0.0)
        f = jnp.maximum(jnp.dot(f, wf2[...],
                                preferred_element_type=jnp.float32)
                        + bf2r[...], 0.0)
        z = jnp.dot(f, wo[...], preferred_element_type=jnp.float32) + bor[...]
        o_ref[...] = jax.nn.sigmoid(z)

    args = [g_raw, xt, Wg1T, bg1.reshape(1, -1), Wg2T, bg2.reshape(1, -1),
            Wf1T, bf1.reshape(1, -1), Wf2T, bf2.reshape(1, -1),
            WoutT, bout.reshape(1, -1)]
    return pl.pallas_call(
        body,
        out_shape=jax.ShapeDtypeStruct((B, 1), jnp.float32),
    )(*args)


def kernel(x, edge_index, batch, x_cell_mut, edge_feat, W_l1, W_r1, b1,
           W_l2, W_r2, b2, W_l3, W_r3, b3, Wg1, bg1, Wg2, bg2, Wc1, bc1,
           Wc2, bc2, Wc3, bc3, Wxt, bxt, Wf1, bf1, Wf2, bf2, Wout, bout):
    src = edge_index[0]
    dst = edge_index[1]

    acc1, deg = _sc_seg_sum(x, src, dst, compute_deg=True)
    h1, inv_deg = _tc_layer1(acc1, deg, x, W_l1.T, W_r1.T, b1)

    acc2 = _sc_seg_sum(h1, src, dst, compute_deg=False)[0]
    h2 = _tc_layer(acc2, inv_deg, h1, W_l2.T, W_r2.T, b2, merge=True)

    acc3 = _sc_seg_sum_halves(h2[:, :128], h2[:, 128:], src, dst)[0]
    h3 = _tc_layer(acc3, inv_deg, h2, W_l3.T, W_r3.T, b3, merge=False)

    g_raw = _tc_segment_max(h3, batch.reshape(N_NODES, 1), B)

    c1 = _tc_conv_phase(x_cell_mut.reshape(B, 4380, 3),
                        _phase_weights(Wc1), jnp.tile(bc1, 3)[None], 4377, 32)
    c2 = _tc_conv_phase(c1.reshape(B, 1459, 96),
                        _phase_weights(Wc2), jnp.tile(bc2, 3)[None], 1456, 64)
    w3T = jnp.transpose(Wc3, (2, 1, 0))
    c3 = _tc_conv_stage(c2, w3T, bc3)

    flat = c3.reshape(B, -1)
    WxtT = jnp.transpose(Wxt.reshape(128, 128, 483), (2, 1, 0)).reshape(61824, 128)
    xt = _tc_proj(flat, WxtT, bxt)

    return _tc_heads(g_raw, xt, Wg1.T, bg1, Wg2.T, bg2, Wf1.T, bf1,
                     Wf2.T, bf2, Wout.T, bout)


# speedup vs baseline: 1.1946x; 1.0231x over previous
import functools

import jax
import jax.numpy as jnp
from jax import lax
from jax.experimental import pallas as pl
from jax.experimental.pallas import tpu as pltpu
from jax.experimental.pallas import tpu_sc as plsc

N_NODES = 10000
N_EDGES = 320000
B = 64

_NC = 2
_NS = 16
_NW = _NC * _NS


def _make_sc_seg_sum(n, w, e, compute_deg, halves):
    C = 128
    SEG = 8
    NSEG = 20 if halves else 10
    P = SEG // 2
    rps = 1000
    n_drain = n // rps

    mesh = plsc.VectorSubcoreMesh(core_axis_name="c", subcore_axis_name="s")
    out_type = [jax.ShapeDtypeStruct((_NC, n, w), jnp.float32)]
    scratch = [
        pltpu.VMEM_SHARED((n + 8, w), jnp.float32),
        pltpu.VMEM((SEG, C), jnp.int32),
        pltpu.VMEM((SEG, C), jnp.int32),
        pltpu.VMEM((C, w), jnp.float32),
        pltpu.VMEM((C, w), jnp.float32),
        pltpu.VMEM((C,), jnp.int32),
        pltpu.SemaphoreType.DMA,
        pltpu.SemaphoreType.DMA,
    ]
    if compute_deg:
        out_type.append(jax.ShapeDtypeStruct((_NC * n,), jnp.float32))
        scratch += [
            pltpu.VMEM_SHARED((n + 8,), jnp.float32),
            pltpu.VMEM((C,), jnp.float32),
            pltpu.VMEM((rps,), jnp.float32),
        ]

    @functools.partial(
        pl.kernel, mesh=mesh, out_type=tuple(out_type), scratch_types=scratch)
    def k(ha_hbm, hb_hbm, src_hbm, dst_hbm, zrow_hbm, zdeg_hbm, *rest):
        if compute_deg:
            (acc_out, deg_out, shared_acc, src_tab, dst_tab, buf0, buf1,
             dst_v, sem0, sem1, shared_deg, ones_v, degbuf) = rest
        else:
            (acc_out, shared_acc, src_tab, dst_tab, buf0, buf1,
             dst_v, sem0, sem1) = rest
        cid = lax.axis_index("c")
        sid = lax.axis_index("s")
        wid = sid * _NC + cid
        tab_id = sid if halves else wid

        @pl.when(sid < n_drain)
        def _():
            pltpu.sync_copy(zrow_hbm, buf0)
            for t in range(7):
                pltpu.sync_copy(buf0,
                                shared_acc.at[pl.ds(sid * rps + t * 128, 128)])
            pltpu.sync_copy(buf0.at[pl.ds(0, 104)],
                            shared_acc.at[pl.ds(sid * rps + 896, 104)])
        if compute_deg:
            @pl.when(sid < n_drain)
            def _():
                pltpu.sync_copy(zdeg_hbm, degbuf)
                pltpu.sync_copy(degbuf, shared_deg.at[pl.ds(sid * rps, rps)])
            for i in range(C // 16):
                ones_v[pl.ds(i * 16, 16)] = jnp.full((16,), 1.0, jnp.float32)
        plsc.subcore_barrier()

        def run_edges(h_hbm):
            def seg(s, carry):
                pltpu.sync_copy(src_hbm.at[tab_id, s], src_tab)
                pltpu.sync_copy(dst_hbm.at[tab_id, s], dst_tab)
                pltpu.async_copy(h_hbm.at[src_tab.at[0]], buf0, sem0)

                def pair(i, carry2):
                    j0 = 2 * i
                    pltpu.async_copy(h_hbm.at[src_tab.at[j0 + 1]], buf1, sem1)
                    for q in range(C // 16):
                        dst_v[pl.ds(q * 16, 16)] = dst_tab[j0,
                                                           pl.ds(q * 16, 16)]
                    pltpu.make_async_copy(h_hbm.at[src_tab.at[j0]], buf0,
                                          sem0).wait()
                    pltpu.sync_copy(buf0, shared_acc.at[dst_v], add=True)
                    if compute_deg:
                        pltpu.sync_copy(ones_v, shared_deg.at[dst_v],
                                        add=True)

                    @pl.when(i < P - 1)
                    def _():
                        pltpu.async_copy(h_hbm.at[src_tab.at[j0 + 2]], buf0,
                                         sem0)

                    for q in range(C // 16):
                        dst_v[pl.ds(q * 16, 16)] = dst_tab[j0 + 1,
                                                           pl.ds(q * 16, 16)]
                    pltpu.make_async_copy(h_hbm.at[src_tab.at[j0 + 1]], buf1,
                                          sem1).wait()
                    pltpu.sync_copy(buf1, shared_acc.at[dst_v], add=True)
                    if compute_deg:
                        pltpu.sync_copy(ones_v, shared_deg.at[dst_v],
                                        add=True)
                    return carry2

                lax.fori_loop(0, P, pair, 0)
                return carry

            lax.fori_loop(0, NSEG, seg, 0)

        if halves:
            @pl.when(cid == 0)
            def _():
                run_edges(ha_hbm)

            @pl.when(cid == 1)
            def _():
                run_edges(hb_hbm)
        else:
            run_edges(ha_hbm)
        plsc.subcore_barrier()

        r0 = sid * rps

        @pl.when(sid < n_drain)
        def _():
            for t in range(7):
                pltpu.sync_copy(shared_acc.at[pl.ds(r0 + t * 128, 128)], buf0)
                pltpu.sync_copy(buf0, acc_out.at[cid, pl.ds(r0 + t * 128, 128)])
            pltpu.sync_copy(shared_acc.at[pl.ds(r0 + 896, 104)],
                            buf0.at[pl.ds(0, 104)])
            pltpu.sync_copy(buf0.at[pl.ds(0, 104)],
                            acc_out.at[cid, pl.ds(r0 + 896, 104)])
            if compute_deg:
                pltpu.sync_copy(shared_deg.at[pl.ds(r0, rps)], degbuf)
                pltpu.sync_copy(degbuf, deg_out.at[pl.ds(cid * n + r0, rps)])

    return k


def _pad_tables(src, dst, n, nworkers, nseg):
    e = src.shape[0]
    epw = e // nworkers
    pad = nseg * 8 * 128 - epw
    s2 = src.reshape(nworkers, epw)
    d2 = dst.reshape(nworkers, epw)
    s2 = jnp.concatenate(
        [s2, jnp.zeros((nworkers, pad), jnp.int32)], axis=1)
    d2 = jnp.concatenate(
        [d2, jnp.full((nworkers, pad), n, jnp.int32)], axis=1)
    return (s2.reshape(nworkers, nseg, 8, 128),
            d2.reshape(nworkers, nseg, 8, 128))


def _sc_seg_sum(h, src, dst, compute_deg):
    n, w = h.shape
    zrow = jnp.zeros((128, w), jnp.float32)
    zdeg = jnp.zeros((1000,), jnp.float32)
    src4, dst4 = _pad_tables(src, dst, n, _NW, 10)
    k = _make_sc_seg_sum(n, w, src.shape[0], compute_deg, halves=False)
    return k(h, h, src4, dst4, zrow, zdeg)


def _sc_seg_sum_halves(ha, hb, src, dst):
    n, w = ha.shape
    zrow = jnp.zeros((128, w), jnp.float32)
    zdeg = jnp.zeros((1000,), jnp.float32)
    src4, dst4 = _pad_tables(src, dst, n, _NS, 20)
    k = _make_sc_seg_sum(n, w, src.shape[0], compute_deg=False, halves=True)
    return k(ha, hb, src4, dst4, zrow, zdeg)


_RB = 400


def _tc_layer1(acc, deg, x, WlT, WrT, b):
    n, w = x.shape
    dout = WlT.shape[1]
    grid = (n // _RB,)

    def body(acc_ref, deg_ref, x_ref, wl_ref, wr_ref, b_ref, h_ref, inv_ref):
        d = jnp.maximum(deg_ref[0] + deg_ref[1], 1.0)
        inv = 1.0 / d
        inv_ref[...] = inv
        agg = (acc_ref[0] + acc_ref[1]) * inv
        h = jnp.dot(agg, wl_ref[...], preferred_element_type=jnp.float32)
        h += jnp.dot(x_ref[...], wr_ref[...], preferred_element_type=jnp.float32)
        h_ref[...] = jnp.maximum(h + b_ref[...], 0.0)

    return pl.pallas_call(
        body,
        grid=grid,
        in_specs=[
            pl.BlockSpec((2, _RB, w), lambda i: (0, i, 0)),
            pl.BlockSpec((2, _RB, 1), lambda i: (0, i, 0)),
            pl.BlockSpec((_RB, w), lambda i: (i, 0)),
            pl.BlockSpec(WlT.shape, lambda i: (0, 0)),
            pl.BlockSpec(WrT.shape, lambda i: (0, 0)),
            pl.BlockSpec((1, dout), lambda i: (0, 0)),
        ],
        out_specs=[
            pl.BlockSpec((_RB, dout), lambda i: (i, 0)),
            pl.BlockSpec((_RB, 1), lambda i: (i, 0)),
        ],
        out_shape=[
            jax.ShapeDtypeStruct((n, dout), jnp.float32),
            jax.ShapeDtypeStruct((n, 1), jnp.float32),
        ],
    )(acc, deg.reshape(2, n, 1), x, WlT, WrT, b.reshape(1, dout))


def _tc_layer(acc, inv_deg, x, WlT, WrT, b, merge):
    n = x.shape[0]
    din = x.shape[1]
    dout = WlT.shape[1]
    w_each = acc.shape[2]
    grid = (n // _RB,)

    def body(acc_ref, inv_ref, x_ref, wl_ref, wr_ref, b_ref, h_ref):
        inv = inv_ref[...]
        if merge:
            agg = (acc_ref[0] + acc_ref[1]) * inv
        else:
            agg = jnp.concatenate([acc_ref[0] * inv, acc_ref[1] * inv], axis=1)
        h = jnp.dot(agg, wl_ref[...], preferred_element_type=jnp.float32)
        h += jnp.dot(x_ref[...], wr_ref[...], preferred_element_type=jnp.float32)
        h_ref[...] = jnp.maximum(h + b_ref[...], 0.0)

    in_specs = [pl.BlockSpec((2, _RB, w_each), lambda i: (0, i, 0))]
    in_specs += [
        pl.BlockSpec((_RB, 1), lambda i: (i, 0)),
        pl.BlockSpec((_RB, din), lambda i: (i, 0)),
        pl.BlockSpec(WlT.shape, lambda i: (0, 0)),
        pl.BlockSpec(WrT.shape, lambda i: (0, 0)),
        pl.BlockSpec((1, dout), lambda i: (0, 0)),
    ]
    return pl.pallas_call(
        body,
        grid=grid,
        in_specs=in_specs,
        out_specs=pl.BlockSpec((_RB, dout), lambda i: (i, 0)),
        out_shape=jax.ShapeDtypeStruct((n, dout), jnp.float32),
    )(acc, inv_deg, x, WlT, WrT, b.reshape(1, dout))


def _tc_segment_max(h, batch2d, nseg):
    n, d = h.shape
    grid = (n // _RB,)

    def body(h_ref, b_ref, g_ref):
        i = pl.program_id(0)

        @pl.when(i == 0)
        def _():
            g_ref[...] = jnp.full((nseg, d), -jnp.inf, jnp.float32)

        bmin = b_ref[0, 0]
        bmax = b_ref[_RB - 1, 0]
        hv = h_ref[...]
        bv = b_ref[...]
        for s in range(nseg):
            @pl.when((s >= bmin) & (s <= bmax))
            def _():
                vals = jnp.where(bv == s, hv, -jnp.inf)
                m = jnp.max(vals, axis=0, keepdims=True)
                g_ref[pl.ds(s, 1), :] = jnp.maximum(g_ref[pl.ds(s, 1), :], m)

    return pl.pallas_call(
        body,
        grid=grid,
        in_specs=[
            pl.BlockSpec((_RB, d), lambda i: (i, 0)),
            pl.BlockSpec((_RB, 1), lambda i: (i, 0)),
        ],
        out_specs=pl.BlockSpec((nseg, d), lambda i: (0, 0)),
        out_shape=jax.ShapeDtypeStruct((nseg, d), jnp.float32),
    )(h, batch2d)


def _phase_weights(w):
    cout, cin, _ = w.shape
    rows = []
    for rr in range(12):
        blocks = []
        for j in range(3):
            t = rr - j
            if 0 <= t < 8:
                blocks.append(w[:, :, t].T)
            else:
                blocks.append(jnp.zeros((cin, cout), jnp.float32))
        rows.append(jnp.concatenate(blocks, axis=1))
    return jnp.concatenate(rows, axis=0)


def _tc_conv_phase(xg, wp, bp, lp, cout):
    _, G, M = xg.shape

    def body(x_ref, w_ref, b_ref, o_ref):
        xr = x_ref[...].reshape(G, M)
        cols = jnp.concatenate([xr[s:s + lp] for s in range(4)], axis=1)
        pre = jnp.dot(cols, w_ref[...], preferred_element_type=jnp.float32)
        pre = jnp.maximum(pre + b_ref[...], 0.0)
        y = jnp.maximum(jnp.maximum(pre[:, :cout], pre[:, cout:2 * cout]),
                        pre[:, 2 * cout:])
        o_ref[...] = y[None]

    return pl.pallas_call(
        body,
        grid=(B,),
        in_specs=[
            pl.BlockSpec((1, G, M), lambda n: (n, 0, 0)),
            pl.BlockSpec(wp.shape, lambda n: (0, 0)),
            pl.BlockSpec((1, 3 * cout), lambda n: (0, 0)),
        ],
        out_specs=pl.BlockSpec((1, lp, cout), lambda n: (n, 0, 0)),
        out_shape=jax.ShapeDtypeStruct((B, lp, cout), jnp.float32),
    )(xg, wp, bp)


def _tc_conv_stage1(x3, w1, b1):
    L = x3.shape[2]
    lo = L - 7
    lp = (lo // 3)
    cout = w1.shape[0]

    def body(x_ref, w_ref, b_ref, o_ref):
        xr = x_ref[...].reshape(1, L)
        cols = jnp.concatenate([xr[:, t:t + lo] for t in range(8)], axis=0)
        pre = jnp.dot(w_ref[...], cols, preferred_element_type=jnp.float32)
        pre = jnp.maximum(pre + b_ref[...], 0.0)
        preT = jnp.transpose(pre, (1, 0))
        preT = preT[:lp * 3].reshape(lp, 3, cout)
        o_ref[...] = jnp.max(preT, axis=1)[None]

    return pl.pallas_call(
        body,
        grid=(B,),
        in_specs=[
            pl.BlockSpec((1, 1, L), lambda n: (n, 0, 0)),
            pl.BlockSpec((cout, 8), lambda n: (0, 0)),
            pl.BlockSpec((cout, 1), lambda n: (0, 0)),
        ],
        out_specs=pl.BlockSpec((1, lp, cout), lambda n: (n, 0, 0)),
        out_shape=jax.ShapeDtypeStruct((B, lp, cout), jnp.float32),
    )(x3, w1, b1.reshape(cout, 1))


def _tc_conv_stage(x, wT, b):
    _, L, cin = x.shape
    cout = wT.shape[2]
    lo = L - 7
    lp = lo // 3

    def body(x_ref, w_ref, b_ref, o_ref):
        xc = x_ref[...].reshape(L, cin)
        pre = b_ref[...]
        for t in range(8):
            pre = pre + jnp.dot(xc[t:t + lo, :], w_ref[t],
                                preferred_element_type=jnp.float32)
        pre = jnp.maximum(pre, 0.0)
        pre = pre[:lp * 3].reshape(lp, 3, cout)
        o_ref[...] = jnp.max(pre, axis=1)[None]

    return pl.pallas_call(
        body,
        grid=(B,),
        in_specs=[
            pl.BlockSpec((1, L, cin), lambda n: (n, 0, 0)),
            pl.BlockSpec(wT.shape, lambda n: (0, 0, 0)),
            pl.BlockSpec((1, cout), lambda n: (0, 0)),
        ],
        out_specs=pl.BlockSpec((1, lp, cout), lambda n: (n, 0, 0)),
        out_shape=jax.ShapeDtypeStruct((B, lp, cout), jnp.float32),
    )(x, wT, b.reshape(1, cout))


def _tc_proj(flat, WxtT, bxt):
    k = flat.shape[1]
    kb = 8832
    steps = k // kb
    dout = WxtT.shape[1]

    def body(a_ref, w_ref, b_ref, o_ref):
        j = pl.program_id(0)

        @pl.when(j == 0)
        def _():
            o_ref[...] = jnp.broadcast_to(b_ref[...], (B, dout))

        o_ref[...] += jnp.dot(a_ref[...], w_ref[...],
                              preferred_element_type=jnp.float32)

    return pl.pallas_call(
        body,
        grid=(steps,),
        in_specs=[
            pl.BlockSpec((B, kb), lambda j: (0, j)),
            pl.BlockSpec((kb, dout), lambda j: (j, 0)),
            pl.BlockSpec((1, dout), lambda j: (0, 0)),
        ],
        out_specs=pl.BlockSpec((B, dout), lambda j: (0, 0)),
        out_shape=jax.ShapeDtypeStruct((B, dout), jnp.float32),
    )(flat, WxtT, bxt.reshape(1, dout))


def _tc_heads(g_raw, xt, Wg1T, bg1, Wg2T, bg2, Wf1T, bf1, Wf2T, bf2,
              WoutT, bout):
    def body(g_ref, xt_ref, wg1, bg1r, wg2, bg2r, wf1, bf1r, wf2, bf2r,
             wo, bor, o_ref):
        g = jnp.maximum(jnp.dot(g_ref[...], wg1[...],
                                preferred_element_type=jnp.float32)
                        + bg1r[...], 0.0)
        g = jnp.dot(g, wg2[...], preferred_element_type=jnp.float32) + bg2r[...]
        xc = jnp.concatenate([g, xt_ref[...]], axis=1)
        f = jnp.maximum(jnp.dot(xc, wf1[...],
                                preferred_element_type=jnp.float32)
                        + bf1r[...], 0.0)
        f = jnp.maximum(jnp.dot(f, wf2[...],
                                preferred_element_type=jnp.float32)
                        + bf2r[...], 0.0)
        z = jnp.dot(f, wo[...], preferred_element_type=jnp.float32) + bor[...]
        o_ref[...] = jax.nn.sigmoid(z)

    args = [g_raw, xt, Wg1T, bg1.reshape(1, -1), Wg2T, bg2.reshape(1, -1),
            Wf1T, bf1.reshape(1, -1), Wf2T, bf2.reshape(1, -1),
            WoutT, bout.reshape(1, -1)]
    return pl.pallas_call(
        body,
        out_shape=jax.ShapeDtypeStruct((B, 1), jnp.float32),
    )(*args)


def kernel(x, edge_index, batch, x_cell_mut, edge_feat, W_l1, W_r1, b1,
           W_l2, W_r2, b2, W_l3, W_r3, b3, Wg1, bg1, Wg2, bg2, Wc1, bc1,
           Wc2, bc2, Wc3, bc3, Wxt, bxt, Wf1, bf1, Wf2, bf2, Wout, bout):
    src = edge_index[0]
    dst = edge_index[1]

    acc1, deg = _sc_seg_sum(x, src, dst, compute_deg=True)
    h1, inv_deg = _tc_layer1(acc1, deg, x, W_l1.T, W_r1.T, b1)

    acc2 = _sc_seg_sum(h1, src, dst, compute_deg=False)[0]
    h2 = _tc_layer(acc2, inv_deg, h1, W_l2.T, W_r2.T, b2, merge=True)

    acc3 = _sc_seg_sum_halves(h2[:, :128], h2[:, 128:], src, dst)[0]
    h3 = _tc_layer(acc3, inv_deg, h2, W_l3.T, W_r3.T, b3, merge=False)

    g_raw = _tc_segment_max(h3, batch.reshape(N_NODES, 1), B)

    c1 = _tc_conv_phase(x_cell_mut.reshape(B, 4380, 3),
                        _phase_weights(Wc1), jnp.tile(bc1, 3)[None], 4377, 32)
    c2 = _tc_conv_phase(c1.reshape(B, 1459, 96),
                        _phase_weights(Wc2), jnp.tile(bc2, 3)[None], 1456, 64)
    w3T = jnp.transpose(Wc3, (2, 1, 0))
    c3 = _tc_conv_stage(c2, w3T, bc3)

    flat = c3.reshape(B, -1)
    WxtT = jnp.transpose(Wxt.reshape(128, 128, 483), (2, 1, 0)).reshape(61824, 128)
    xt = _tc_proj(flat, WxtT, bxt)

    return _tc_heads(g_raw, xt, Wg1.T, bg1, Wg2.T, bg2, Wf1.T, bf1,
                     Wf2.T, bf2, Wout.T, bout)


# speedup vs baseline: 1.1994x; 1.0040x over previous
import functools

import jax
import jax.numpy as jnp
from jax import lax
from jax.experimental import pallas as pl
from jax.experimental.pallas import tpu as pltpu
from jax.experimental.pallas import tpu_sc as plsc

N_NODES = 10000
N_EDGES = 320000
B = 64

_NC = 2
_NS = 16
_NW = _NC * _NS


def _make_sc_seg_sum(n, w, e, compute_deg, halves):
    C = 128
    SEG = 8
    NSEG = 20 if halves else 10
    P = SEG // 2
    rps = 1000
    n_drain = n // rps

    mesh = plsc.VectorSubcoreMesh(core_axis_name="c", subcore_axis_name="s")
    out_type = [jax.ShapeDtypeStruct((_NC, n, w), jnp.float32)]
    scratch = [
        pltpu.VMEM_SHARED((n + 8, w), jnp.float32),
        pltpu.VMEM((SEG, C), jnp.int32),
        pltpu.VMEM((SEG, C), jnp.int32),
        pltpu.VMEM((C, w), jnp.float32),
        pltpu.VMEM((C, w), jnp.float32),
        pltpu.VMEM((C,), jnp.int32),
        pltpu.SemaphoreType.DMA,
        pltpu.SemaphoreType.DMA,
    ]
    if compute_deg:
        out_type.append(jax.ShapeDtypeStruct((_NC * n,), jnp.float32))
        scratch += [
            pltpu.VMEM_SHARED((n + 8,), jnp.float32),
            pltpu.VMEM((C,), jnp.float32),
            pltpu.VMEM((rps,), jnp.float32),
        ]

    @functools.partial(
        pl.kernel, mesh=mesh, out_type=tuple(out_type), scratch_types=scratch)
    def k(ha_hbm, hb_hbm, src_hbm, dst_hbm, zrow_hbm, zdeg_hbm, *rest):
        if compute_deg:
            (acc_out, deg_out, shared_acc, src_tab, dst_tab, buf0, buf1,
             dst_v, sem0, sem1, shared_deg, ones_v, degbuf) = rest
        else:
            (acc_out, shared_acc, src_tab, dst_tab, buf0, buf1,
             dst_v, sem0, sem1) = rest
        cid = lax.axis_index("c")
        sid = lax.axis_index("s")
        wid = sid * _NC + cid
        tab_id = sid if halves else wid

        @pl.when(sid < n_drain)
        def _():
            pltpu.sync_copy(zrow_hbm, buf0)
            for t in range(7):
                pltpu.sync_copy(buf0,
                                shared_acc.at[pl.ds(sid * rps + t * 128, 128)])
            pltpu.sync_copy(buf0.at[pl.ds(0, 104)],
                            shared_acc.at[pl.ds(sid * rps + 896, 104)])
        if compute_deg:
            @pl.when(sid < n_drain)
            def _():
                pltpu.sync_copy(zdeg_hbm, degbuf)
                pltpu.sync_copy(degbuf, shared_deg.at[pl.ds(sid * rps, rps)])
            for i in range(C // 16):
                ones_v[pl.ds(i * 16, 16)] = jnp.full((16,), 1.0, jnp.float32)
        plsc.subcore_barrier()

        def run_edges(h_hbm):
            def seg(s, carry):
                pltpu.sync_copy(src_hbm.at[tab_id, s], src_tab)
                pltpu.sync_copy(dst_hbm.at[tab_id, s], dst_tab)
                pltpu.async_copy(h_hbm.at[src_tab.at[0]], buf0, sem0)

                def pair(i, carry2):
                    j0 = 2 * i
                    pltpu.async_copy(h_hbm.at[src_tab.at[j0 + 1]], buf1, sem1)
                    for q in range(C // 16):
                        dst_v[pl.ds(q * 16, 16)] = dst_tab[j0,
                                                           pl.ds(q * 16, 16)]
                    pltpu.make_async_copy(h_hbm.at[src_tab.at[j0]], buf0,
                                          sem0).wait()
                    pltpu.sync_copy(buf0, shared_acc.at[dst_v], add=True)
                    if compute_deg:
                        pltpu.sync_copy(ones_v, shared_deg.at[dst_v],
                                        add=True)

                    @pl.when(i < P - 1)
                    def _():
                        pltpu.async_copy(h_hbm.at[src_tab.at[j0 + 2]], buf0,
                                         sem0)

                    for q in range(C // 16):
                        dst_v[pl.ds(q * 16, 16)] = dst_tab[j0 + 1,
                                                           pl.ds(q * 16, 16)]
                    pltpu.make_async_copy(h_hbm.at[src_tab.at[j0 + 1]], buf1,
                                          sem1).wait()
                    pltpu.sync_copy(buf1, shared_acc.at[dst_v], add=True)
                    if compute_deg:
                        pltpu.sync_copy(ones_v, shared_deg.at[dst_v],
                                        add=True)
                    return carry2

                lax.fori_loop(0, P, pair, 0)
                return carry

            lax.fori_loop(0, NSEG, seg, 0)

        if halves:
            @pl.when(cid == 0)
            def _():
                run_edges(ha_hbm)

            @pl.when(cid == 1)
            def _():
                run_edges(hb_hbm)
        else:
            run_edges(ha_hbm)
        plsc.subcore_barrier()

        r0 = sid * rps

        @pl.when(sid < n_drain)
        def _():
            for t in range(7):
                pltpu.sync_copy(shared_acc.at[pl.ds(r0 + t * 128, 128)], buf0)
                pltpu.sync_copy(buf0, acc_out.at[cid, pl.ds(r0 + t * 128, 128)])
            pltpu.sync_copy(shared_acc.at[pl.ds(r0 + 896, 104)],
                            buf0.at[pl.ds(0, 104)])
            pltpu.sync_copy(buf0.at[pl.ds(0, 104)],
                            acc_out.at[cid, pl.ds(r0 + 896, 104)])
            if compute_deg:
                pltpu.sync_copy(shared_deg.at[pl.ds(r0, rps)], degbuf)
                pltpu.sync_copy(degbuf, deg_out.at[pl.ds(cid * n + r0, rps)])

    return k


def _pad_tables(src, dst, n, nworkers, nseg):
    e = src.shape[0]
    epw = e // nworkers
    pad = nseg * 8 * 128 - epw
    s2 = src.reshape(nworkers, epw)
    d2 = dst.reshape(nworkers, epw)
    s2 = jnp.concatenate(
        [s2, jnp.zeros((nworkers, pad), jnp.int32)], axis=1)
    d2 = jnp.concatenate(
        [d2, jnp.full((nworkers, pad), n, jnp.int32)], axis=1)
    return (s2.reshape(nworkers, nseg, 8, 128),
            d2.reshape(nworkers, nseg, 8, 128))


def _sc_seg_sum(h, src, dst, compute_deg):
    n, w = h.shape
    zrow = jnp.zeros((128, w), jnp.float32)
    zdeg = jnp.zeros((1000,), jnp.float32)
    src4, dst4 = _pad_tables(src, dst, n, _NW, 10)
    k = _make_sc_seg_sum(n, w, src.shape[0], compute_deg, halves=False)
    return k(h, h, src4, dst4, zrow, zdeg)


def _sc_seg_sum_halves(ha, hb, src, dst):
    n, w = ha.shape
    zrow = jnp.zeros((128, w), jnp.float32)
    zdeg = jnp.zeros((1000,), jnp.float32)
    src4, dst4 = _pad_tables(src, dst, n, _NS, 20)
    k = _make_sc_seg_sum(n, w, src.shape[0], compute_deg=False, halves=True)
    return k(ha, hb, src4, dst4, zrow, zdeg)


_RB = 400


def _tc_layer1(acc, deg, x, WlT, WrT, b):
    n, w = x.shape
    dout = WlT.shape[1]
    grid = (n // _RB,)

    def body(acc_ref, deg_ref, x_ref, wl_ref, wr_ref, b_ref, h_ref, inv_ref):
        d = jnp.maximum(deg_ref[0] + deg_ref[1], 1.0)
        inv = 1.0 / d
        inv_ref[...] = inv
        agg = (acc_ref[0] + acc_ref[1]) * inv
        h = jnp.dot(agg, wl_ref[...], preferred_element_type=jnp.float32)
        h += jnp.dot(x_ref[...], wr_ref[...], preferred_element_type=jnp.float32)
        h_ref[...] = jnp.maximum(h + b_ref[...], 0.0)

    return pl.pallas_call(
        body,
        grid=grid,
        in_specs=[
            pl.BlockSpec((2, _RB, w), lambda i: (0, i, 0)),
            pl.BlockSpec((2, _RB, 1), lambda i: (0, i, 0)),
            pl.BlockSpec((_RB, w), lambda i: (i, 0)),
            pl.BlockSpec(WlT.shape, lambda i: (0, 0)),
            pl.BlockSpec(WrT.shape, lambda i: (0, 0)),
            pl.BlockSpec((1, dout), lambda i: (0, 0)),
        ],
        out_specs=[
            pl.BlockSpec((_RB, dout), lambda i: (i, 0)),
            pl.BlockSpec((_RB, 1), lambda i: (i, 0)),
        ],
        out_shape=[
            jax.ShapeDtypeStruct((n, dout), jnp.float32),
            jax.ShapeDtypeStruct((n, 1), jnp.float32),
        ],
    )(acc, deg.reshape(2, n, 1), x, WlT, WrT, b.reshape(1, dout))


def _tc_layer(acc, inv_deg, x, WlT, WrT, b, merge):
    n = x.shape[0]
    din = x.shape[1]
    dout = WlT.shape[1]
    w_each = acc.shape[2]
    grid = (n // _RB,)

    def body(acc_ref, inv_ref, x_ref, wl_ref, wr_ref, b_ref, h_ref):
        inv = inv_ref[...]
        if merge:
            agg = (acc_ref[0] + acc_ref[1]) * inv
        else:
            agg = jnp.concatenate([acc_ref[0] * inv, acc_ref[1] * inv], axis=1)
        h = jnp.dot(agg, wl_ref[...], preferred_element_type=jnp.float32)
        h += jnp.dot(x_ref[...], wr_ref[...], preferred_element_type=jnp.float32)
        h_ref[...] = jnp.maximum(h + b_ref[...], 0.0)

    in_specs = [pl.BlockSpec((2, _RB, w_each), lambda i: (0, i, 0))]
    in_specs += [
        pl.BlockSpec((_RB, 1), lambda i: (i, 0)),
        pl.BlockSpec((_RB, din), lambda i: (i, 0)),
        pl.BlockSpec(WlT.shape, lambda i: (0, 0)),
        pl.BlockSpec(WrT.shape, lambda i: (0, 0)),
        pl.BlockSpec((1, dout), lambda i: (0, 0)),
    ]
    return pl.pallas_call(
        body,
        grid=grid,
        in_specs=in_specs,
        out_specs=pl.BlockSpec((_RB, dout), lambda i: (i, 0)),
        out_shape=jax.ShapeDtypeStruct((n, dout), jnp.float32),
    )(acc, inv_deg, x, WlT, WrT, b.reshape(1, dout))


def _tc_segment_max(h, batch2d, nseg):
    n, d = h.shape
    grid = (n // _RB,)

    def body(h_ref, b_ref, g_ref):
        i = pl.program_id(0)

        @pl.when(i == 0)
        def _():
            g_ref[...] = jnp.full((nseg, d), -jnp.inf, jnp.float32)

        bmin = b_ref[0, 0]
        bmax = b_ref[_RB - 1, 0]
        hv = h_ref[...]
        bv = b_ref[...]

        def seg_body(s, carry):
            vals = jnp.where(bv == s, hv, -jnp.inf)
            m = jnp.max(vals, axis=0, keepdims=True)
            g_ref[pl.ds(s, 1), :] = jnp.maximum(g_ref[pl.ds(s, 1), :], m)
            return carry

        lax.fori_loop(bmin, bmax + 1, seg_body, 0)

    return pl.pallas_call(
        body,
        grid=grid,
        in_specs=[
            pl.BlockSpec((_RB, d), lambda i: (i, 0)),
            pl.BlockSpec((_RB, 1), lambda i: (i, 0)),
        ],
        out_specs=pl.BlockSpec((nseg, d), lambda i: (0, 0)),
        out_shape=jax.ShapeDtypeStruct((nseg, d), jnp.float32),
    )(h, batch2d)


def _phase_weights(w):
    cout, cin, _ = w.shape
    rows = []
    for rr in range(12):
        blocks = []
        for j in range(3):
            t = rr - j
            if 0 <= t < 8:
                blocks.append(w[:, :, t].T)
            else:
                blocks.append(jnp.zeros((cin, cout), jnp.float32))
        rows.append(jnp.concatenate(blocks, axis=1))
    return jnp.concatenate(rows, axis=0)


def _tc_conv_phase(xg, wp, bp, lp, cout):
    _, G, M = xg.shape

    def body(x_ref, w_ref, b_ref, o_ref):
        xr = x_ref[...].reshape(G, M)
        cols = jnp.concatenate([xr[s:s + lp] for s in range(4)], axis=1)
        pre = jnp.dot(cols, w_ref[...], preferred_element_type=jnp.float32)
        pre = jnp.maximum(pre + b_ref[...], 0.0)
        y = jnp.maximum(jnp.maximum(pre[:, :cout], pre[:, cout:2 * cout]),
                        pre[:, 2 * cout:])
        o_ref[...] = y[None]

    return pl.pallas_call(
        body,
        grid=(B,),
        in_specs=[
            pl.BlockSpec((1, G, M), lambda n: (n, 0, 0)),
            pl.BlockSpec(wp.shape, lambda n: (0, 0)),
            pl.BlockSpec((1, 3 * cout), lambda n: (0, 0)),
        ],
        out_specs=pl.BlockSpec((1, lp, cout), lambda n: (n, 0, 0)),
        out_shape=jax.ShapeDtypeStruct((B, lp, cout), jnp.float32),
    )(xg, wp, bp)


def _tc_conv_stage(x, wT, b):
    _, L, cin = x.shape
    cout = wT.shape[2]
    lo = L - 7
    lp = lo // 3

    def body(x_ref, w_ref, b_ref, o_ref):
        xc = x_ref[...].reshape(L, cin)
        pre = b_ref[...]
        for t in range(8):
            pre = pre + jnp.dot(xc[t:t + lo, :], w_ref[t],
                                preferred_element_type=jnp.float32)
        pre = jnp.maximum(pre, 0.0)
        pre = pre[:lp * 3].reshape(lp, 3, cout)
        o_ref[...] = jnp.max(pre, axis=1)[None]

    return pl.pallas_call(
        body,
        grid=(B,),
        in_specs=[
            pl.BlockSpec((1, L, cin), lambda n: (n, 0, 0)),
            pl.BlockSpec(wT.shape, lambda n: (0, 0, 0)),
            pl.BlockSpec((1, cout), lambda n: (0, 0)),
        ],
        out_specs=pl.BlockSpec((1, lp, cout), lambda n: (n, 0, 0)),
        out_shape=jax.ShapeDtypeStruct((B, lp, cout), jnp.float32),
    )(x, wT, b.reshape(1, cout))


def _tc_proj(flat, WxtT, bxt):
    k = flat.shape[1]
    kb = 8832
    steps = k // kb
    dout = WxtT.shape[1]

    def body(a_ref, w_ref, b_ref, o_ref):
        j = pl.program_id(0)

        @pl.when(j == 0)
        def _():
            o_ref[...] = jnp.broadcast_to(b_ref[...], (B, dout))

        o_ref[...] += jnp.dot(a_ref[...], w_ref[...],
                              preferred_element_type=jnp.float32)

    return pl.pallas_call(
        body,
        grid=(steps,),
        in_specs=[
            pl.BlockSpec((B, kb), lambda j: (0, j)),
            pl.BlockSpec((kb, dout), lambda j: (j, 0)),
            pl.BlockSpec((1, dout), lambda j: (0, 0)),
        ],
        out_specs=pl.BlockSpec((B, dout), lambda j: (0, 0)),
        out_shape=jax.ShapeDtypeStruct((B, dout), jnp.float32),
    )(flat, WxtT, bxt.reshape(1, dout))


def _tc_heads(g_raw, xt, Wg1T, bg1, Wg2T, bg2, Wf1T, bf1, Wf2T, bf2,
              WoutT, bout):
    def body(g_ref, xt_ref, wg1, bg1r, wg2, bg2r, wf1, bf1r, wf2, bf2r,
             wo, bor, o_ref):
        g = jnp.maximum(jnp.dot(g_ref[...], wg1[...],
                                preferred_element_type=jnp.float32)
                        + bg1r[...], 0.0)
        g = jnp.dot(g, wg2[...], preferred_element_type=jnp.float32) + bg2r[...]
        xc = jnp.concatenate([g, xt_ref[...]], axis=1)
        f = jnp.maximum(jnp.dot(xc, wf1[...],
                                preferred_element_type=jnp.float32)
                        + bf1r[...], 0.0)
        f = jnp.maximum(jnp.dot(f, wf2[...],
                                preferred_element_type=jnp.float32)
                        + bf2r[...], 0.0)
        z = jnp.dot(f, wo[...], preferred_element_type=jnp.float32) + bor[...]
        o_ref[...] = jax.nn.sigmoid(z)

    args = [g_raw, xt, Wg1T, bg1.reshape(1, -1), Wg2T, bg2.reshape(1, -1),
            Wf1T, bf1.reshape(1, -1), Wf2T, bf2.reshape(1, -1),
            WoutT, bout.reshape(1, -1)]
    return pl.pallas_call(
        body,
        out_shape=jax.ShapeDtypeStruct((B, 1), jnp.float32),
    )(*args)


def kernel(x, edge_index, batch, x_cell_mut, edge_feat, W_l1, W_r1, b1,
           W_l2, W_r2, b2, W_l3, W_r3, b3, Wg1, bg1, Wg2, bg2, Wc1, bc1,
           Wc2, bc2, Wc3, bc3, Wxt, bxt, Wf1, bf1, Wf2, bf2, Wout, bout):
    src = edge_index[0]
    dst = edge_index[1]

    acc1, deg = _sc_seg_sum(x, src, dst, compute_deg=True)
    h1, inv_deg = _tc_layer1(acc1, deg, x, W_l1.T, W_r1.T, b1)

    acc2 = _sc_seg_sum(h1, src, dst, compute_deg=False)[0]
    h2 = _tc_layer(acc2, inv_deg, h1, W_l2.T, W_r2.T, b2, merge=True)

    acc3 = _sc_seg_sum_halves(h2[:, :128], h2[:, 128:], src, dst)[0]
    h3 = _tc_layer(acc3, inv_deg, h2, W_l3.T, W_r3.T, b3, merge=False)

    g_raw = _tc_segment_max(h3, batch.reshape(N_NODES, 1), B)

    c1 = _tc_conv_phase(x_cell_mut.reshape(B, 4380, 3),
                        _phase_weights(Wc1), jnp.tile(bc1, 3)[None], 4377, 32)
    c2 = _tc_conv_phase(c1.reshape(B, 1459, 96),
                        _phase_weights(Wc2), jnp.tile(bc2, 3)[None], 1456, 64)
    w3T = jnp.transpose(Wc3, (2, 1, 0))
    c3 = _tc_conv_stage(c2, w3T, bc3)

    flat = c3.reshape(B, -1)
    WxtT = jnp.transpose(Wxt.reshape(128, 128, 483), (2, 1, 0)).reshape(61824, 128)
    xt = _tc_proj(flat, WxtT, bxt)

    return _tc_heads(g_raw, xt, Wg1.T, bg1, Wg2.T, bg2, Wf1.T, bf1,
                     Wf2.T, bf2, Wout.T, bout)
